# baseline (speedup 1.0000x reference)
import jax
import jax.numpy as jnp
from jax.experimental import pallas as pl
from jax.experimental.pallas import tpu as pltpu


def _rnn_body(CH, I, H, nc, KP, G, SB):
    def body(x_ref, len_ref, wcat_ref, wl_ref, bl_ref, o_ref, h_ref):
        c = pl.program_id(1)

        @pl.when(c == 0)
        def _():
            h_ref[...] = jnp.zeros_like(h_ref)

        lens = len_ref[...]
        hs = [h_ref[:, g * SB:(g + 1) * SB] for g in range(G)]
        lns = [lens[:, g * SB:(g + 1) * SB] for g in range(G)]
        xs = x_ref[...]
        for k in range(CH):
            t = c * CH + k
            for g in range(G):
                xt = xs[k, :, g * SB:(g + 1) * SB]
                rhs = jnp.concatenate([hs[g], xt], axis=0)
                z = jnp.dot(wcat_ref[t], rhs,
                            preferred_element_type=jnp.float32)
                nh = jnp.tanh(z).astype(jnp.bfloat16)
                hs[g] = jnp.where(lns[g] > t, nh, hs[g])
        h = jnp.concatenate(hs, axis=1)
        h_ref[...] = h

        @pl.when(c == nc - 1)
        def _():
            ht = h.T
            o_ref[...] = (jnp.dot(ht, wl_ref[...],
                                  preferred_element_type=jnp.float32)
                          + bl_ref[...])

    return body


def kernel(padded_batch, lengths, W_xh, W_hh, W_lin, b_lin):
    B, T, I = padded_batch.shape
    H = W_hh.shape[-1]
    OUT = W_lin.shape[-1]

    BB = 2048 if B % 2048 == 0 else B
    CH = 64 if T % 64 == 0 else T
    nb = B // BB
    nc = T // CH
    G = 4 if BB % (4 * 128) == 0 else 1
    SB = BB // G

    KP = H + I

    x_t3 = jnp.transpose(padded_batch, (1, 2, 0)).astype(jnp.bfloat16)

    wcat = jnp.concatenate(
        [jnp.transpose(W_hh, (0, 2, 1)),
         jnp.transpose(W_xh, (0, 2, 1))],
        axis=2).astype(jnp.bfloat16)

    lens2 = lengths.astype(jnp.int32).reshape(1, B)
    wl2 = W_lin.astype(jnp.bfloat16)
    bl2 = b_lin.reshape(1, OUT).astype(jnp.float32)

    out = pl.pallas_call(
        _rnn_body(CH, I, H, nc, KP, G, SB),
        out_shape=jax.ShapeDtypeStruct((B, OUT), jnp.float32),
        grid=(nb, nc),
        in_specs=[
            pl.BlockSpec((CH, I, BB), lambda i, c: (c, 0, i)),
            pl.BlockSpec((1, BB), lambda i, c: (0, i)),
            pl.BlockSpec((T, H, KP), lambda i, c: (0, 0, 0)),
            pl.BlockSpec((H, OUT), lambda i, c: (0, 0)),
            pl.BlockSpec((1, OUT), lambda i, c: (0, 0)),
        ],
        out_specs=pl.BlockSpec((BB, OUT), lambda i, c: (i, 0)),
        scratch_shapes=[pltpu.VMEM((H, BB), jnp.bfloat16)],
        compiler_params=pltpu.CompilerParams(
            dimension_semantics=("parallel", "arbitrary"),
        ),
        name="padded_rnn",
    )(x_t3, lens2, wcat, wl2, bl2)
    return out


# speedup vs baseline: 1.1192x; 1.1192x over previous
import jax
import jax.numpy as jnp
from jax.experimental import pallas as pl
from jax.experimental.pallas import tpu as pltpu


def _rnn_body(CH, I, H, nc, G, SB):
    def body(x_ref, len_ref, wcat_ref, wl_ref, bl_ref, o_ref, h_ref):
        c = pl.program_id(1)

        @pl.when(c == 0)
        def _():
            h_ref[...] = jnp.zeros_like(h_ref)

        lens = len_ref[...]
        hs = [h_ref[:, g * SB:(g + 1) * SB] for g in range(G)]
        lns = [lens[:, g * SB:(g + 1) * SB] for g in range(G)]
        xs = x_ref[...]
        for k in range(CH):
            t = c * CH + k
            for g in range(G):
                xt = xs[k * I:(k + 1) * I, g * SB:(g + 1) * SB]
                rhs = jnp.concatenate([hs[g], xt], axis=0)
                z = jnp.dot(wcat_ref[t], rhs,
                            preferred_element_type=jnp.float32)
                nh = jnp.tanh(z).astype(jnp.bfloat16)
                hs[g] = jnp.where(lns[g] > t, nh, hs[g])
        h = jnp.concatenate(hs, axis=1)
        h_ref[...] = h

        @pl.when(c == nc - 1)
        def _():
            o_ref[...] = (jnp.dot(h.T, wl_ref[...],
                                  preferred_element_type=jnp.float32)
                          + bl_ref[...])

    return body


def kernel(padded_batch, lengths, W_xh, W_hh, W_lin, b_lin):
    B, T, I = padded_batch.shape
    H = W_hh.shape[-1]
    OUT = W_lin.shape[-1]

    BB = 4096 if B % 4096 == 0 else B
    CH = 64 if T % 64 == 0 else T
    nb = B // BB
    nc = T // CH
    G = 8 if BB % (8 * 128) == 0 else 1
    SB = BB // G

    x2d = jnp.transpose(padded_batch, (1, 2, 0)).astype(jnp.bfloat16)
    x2d = x2d.reshape(T * I, B)

    wcat = jnp.concatenate(
        [jnp.transpose(W_hh, (0, 2, 1)),
         jnp.transpose(W_xh, (0, 2, 1))],
        axis=2).astype(jnp.bfloat16)

    lens2 = lengths.astype(jnp.int32).reshape(1, B)
    wl2 = W_lin.astype(jnp.bfloat16)
    bl2 = b_lin.reshape(1, OUT).astype(jnp.float32)

    out = pl.pallas_call(
        _rnn_body(CH, I, H, nc, G, SB),
        out_shape=jax.ShapeDtypeStruct((B, OUT), jnp.float32),
        grid=(nb, nc),
        in_specs=[
            pl.BlockSpec((CH * I, BB), lambda i, c: (c, i)),
            pl.BlockSpec((1, BB), lambda i, c: (0, i)),
            pl.BlockSpec((T, H, H + I), lambda i, c: (0, 0, 0)),
            pl.BlockSpec((H, OUT), lambda i, c: (0, 0)),
            pl.BlockSpec((1, OUT), lambda i, c: (0, 0)),
        ],
        out_specs=pl.BlockSpec((BB, OUT), lambda i, c: (i, 0)),
        scratch_shapes=[pltpu.VMEM((H, BB), jnp.bfloat16)],
        compiler_params=pltpu.CompilerParams(
            dimension_semantics=("parallel", "arbitrary"),
        ),
        name="padded_rnn",
    )(x2d, lens2, wcat, wl2, bl2)
    return out


# speedup vs baseline: 1.2517x; 1.1184x over previous
import jax
import jax.numpy as jnp
from jax.experimental import pallas as pl
from jax.experimental.pallas import tpu as pltpu


def _rnn_body(CH, IP, H, nc, G, SB):
    def body(x_ref, len_ref, wcat_ref, wl_ref, bl_ref, o_ref, h_ref):
        c = pl.program_id(1)

        @pl.when(c == 0)
        def _():
            h_ref[...] = jnp.zeros_like(h_ref)

        lens = len_ref[...]
        hs = [h_ref[:, g * SB:(g + 1) * SB] for g in range(G)]
        lns = [lens[:, g * SB:(g + 1) * SB] for g in range(G)]
        xs = x_ref[...]
        for k in range(CH):
            t = c * CH + k
            for g in range(G):
                xt = xs[k, :, g * SB:(g + 1) * SB]
                rhs = jnp.concatenate([hs[g], xt], axis=0)
                z = jnp.dot(wcat_ref[t], rhs,
                            preferred_element_type=jnp.float32)
                nh = jnp.tanh(z).astype(jnp.bfloat16)
                hs[g] = jnp.where(lns[g] > t, nh, hs[g])
        h = jnp.concatenate(hs, axis=1)
        h_ref[...] = h

        @pl.when(c == nc - 1)
        def _():
            o_ref[...] = (jnp.dot(h.T, wl_ref[...],
                                  preferred_element_type=jnp.float32)
                          + bl_ref[...])

    return body


def kernel(padded_batch, lengths, W_xh, W_hh, W_lin, b_lin):
    B, T, I = padded_batch.shape
    H = W_hh.shape[-1]
    OUT = W_lin.shape[-1]

    BB = 4096 if B % 4096 == 0 else B
    CH = 64 if T % 64 == 0 else T
    nb = B // BB
    nc = T // CH
    G = 8 if BB % (8 * 128) == 0 else 1
    SB = BB // G

    IP = 16 if I <= 16 else I
    x_t3 = jnp.transpose(padded_batch, (1, 2, 0))
    if IP != I:
        x_t3 = jnp.concatenate(
            [x_t3, jnp.zeros((T, IP - I, B), x_t3.dtype)], axis=1)
    x_t3 = x_t3.astype(jnp.bfloat16)

    wcat = jnp.concatenate(
        [jnp.transpose(W_hh, (0, 2, 1)),
         jnp.transpose(W_xh, (0, 2, 1)),
         jnp.zeros((T, H, IP - I), W_xh.dtype)],
        axis=2).astype(jnp.bfloat16)

    lens2 = lengths.astype(jnp.int32).reshape(1, B)
    wl2 = W_lin.astype(jnp.bfloat16)
    bl2 = b_lin.reshape(1, OUT).astype(jnp.float32)

    out = pl.pallas_call(
        _rnn_body(CH, IP, H, nc, G, SB),
        out_shape=jax.ShapeDtypeStruct((B, OUT), jnp.float32),
        grid=(nb, nc),
        in_specs=[
            pl.BlockSpec((CH, IP, BB), lambda i, c: (c, 0, i)),
            pl.BlockSpec((1, BB), lambda i, c: (0, i)),
            pl.BlockSpec((T, H, H + IP), lambda i, c: (0, 0, 0)),
            pl.BlockSpec((H, OUT), lambda i, c: (0, 0)),
            pl.BlockSpec((1, OUT), lambda i, c: (0, 0)),
        ],
        out_specs=pl.BlockSpec((BB, OUT), lambda i, c: (i, 0)),
        scratch_shapes=[pltpu.VMEM((H, BB), jnp.bfloat16)],
        compiler_params=pltpu.CompilerParams(
            dimension_semantics=("parallel", "arbitrary"),
        ),
        name="padded_rnn",
    )(x_t3, lens2, wcat, wl2, bl2)
    return out


# speedup vs baseline: 1.5728x; 1.2565x over previous
import jax
import jax.numpy as jnp
from jax.experimental import pallas as pl
from jax.experimental.pallas import tpu as pltpu


def _rnn_body(CH, IP, H, nc, G, SB):
    def body(x_ref, len_ref, wcat_ref, wl_ref, bl_ref, o_ref, h_ref):
        c = pl.program_id(1)

        @pl.when(c == 0)
        def _():
            h_ref[...] = jnp.zeros_like(h_ref)

        lens = len_ref[...]
        hs = [h_ref[:, g * SB:(g + 1) * SB] for g in range(G)]
        lns = [lens[:, g * SB:(g + 1) * SB] for g in range(G)]
        xs = x_ref[...]
        for k in range(CH):
            t = c * CH + k
            for g in range(G):
                xt = xs[k, :, g * SB:(g + 1) * SB]
                rhs = jnp.concatenate([hs[g], xt], axis=0)
                z = jnp.dot(wcat_ref[t], rhs,
                            preferred_element_type=jnp.float32)
                nh = jnp.tanh(z).astype(jnp.bfloat16)
                hs[g] = jnp.where(lns[g] > t, nh, hs[g])
        h = jnp.concatenate(hs, axis=1)
        h_ref[...] = h

        @pl.when(c == nc - 1)
        def _():
            o_ref[...] = (jnp.dot(h.T, wl_ref[...],
                                  preferred_element_type=jnp.float32)
                          + bl_ref[...])

    return body


def kernel(padded_batch, lengths, W_xh, W_hh, W_lin, b_lin):
    B, T, I = padded_batch.shape
    H = W_hh.shape[-1]
    OUT = W_lin.shape[-1]

    BB = 4096 if B % 4096 == 0 else B
    CH = 64 if T % 64 == 0 else T
    nb = B // BB
    nc = T // CH
    G = 8 if BB % (8 * 128) == 0 else 1
    SB = BB // G

    IP = I
    x_t3 = jnp.transpose(padded_batch, (1, 2, 0)).astype(jnp.bfloat16)

    wcat = jnp.concatenate(
        [jnp.transpose(W_hh, (0, 2, 1)),
         jnp.transpose(W_xh, (0, 2, 1))],
        axis=2).astype(jnp.bfloat16)

    lens2 = lengths.astype(jnp.int32).reshape(1, B)
    wl2 = W_lin.astype(jnp.bfloat16)
    bl2 = b_lin.reshape(1, OUT).astype(jnp.float32)

    out = pl.pallas_call(
        _rnn_body(CH, IP, H, nc, G, SB),
        out_shape=jax.ShapeDtypeStruct((B, OUT), jnp.float32),
        grid=(nb, nc),
        in_specs=[
            pl.BlockSpec((CH, IP, BB), lambda i, c: (c, 0, i)),
            pl.BlockSpec((1, BB), lambda i, c: (0, i)),
            pl.BlockSpec((T, H, H + IP), lambda i, c: (0, 0, 0)),
            pl.BlockSpec((H, OUT), lambda i, c: (0, 0)),
            pl.BlockSpec((1, OUT), lambda i, c: (0, 0)),
        ],
        out_specs=pl.BlockSpec((BB, OUT), lambda i, c: (i, 0)),
        scratch_shapes=[pltpu.VMEM((H, BB), jnp.bfloat16)],
        compiler_params=pltpu.CompilerParams(
            dimension_semantics=("parallel", "arbitrary"),
        ),
        name="padded_rnn",
    )(x_t3, lens2, wcat, wl2, bl2)
    return out
